# Initial kernel scaffold; baseline (speedup 1.0000x reference)
import jax
import jax.numpy as jnp
from jax.experimental import pallas as pl


def kernel(x, edge_index, edge_weight, Wp, bp, W_init, W_root, b_arma, W_l, b_l, W_r):
    raise NotImplementedError("write your pallas kernel here")


# speedup vs baseline: 7.9570x; 7.9570x over previous
import functools

import jax
import jax.numpy as jnp
from jax import lax
from jax.experimental import pallas as pl
from jax.experimental.pallas import tpu as pltpu
from jax.experimental.pallas import tpu_sc as plsc

N, E, CUR, HID, OUT = 10000, 320000, 128, 128, 128
NC, NS, L = 2, 16, 16
NPAD = 10240
ROWS_PT = NPAD // NS
EP = E // NS
B = 128
NB = EP // B
TAIL = EP - NB * B

def _rsqrt16(d):
  yi = 0x5F3759DF - lax.shift_right_logical(
      lax.bitcast_convert_type(d, jnp.int32), 1)
  y = lax.bitcast_convert_type(yi, jnp.float32)
  for _ in range(3):
    y = y * (1.5 - 0.5 * d * y * y)
  return jnp.where(d > 0.0, y, 0.0)


def _sc_body(row_h, col_h, ew_h, t_h, h_h,
             aggp_h, aggs_h, cnt_h, dinv_h,
             rows_v, rows_t, ridx, cidx, ewb, scl,
             ridx_t, cidx_t, ewb_t, scl_t,
             dloc, nbuf, acc_sh, deg_sh, dinv_sh, sem):
  c = lax.axis_index("c")
  s = lax.axis_index("s")
  zero16 = jnp.zeros((L,), jnp.float32)
  one16 = jnp.ones((L,), jnp.float32)

  def _z2(i, _):
    for j in range(8):
      rows_v[i, pl.ds(j * L, L)] = zero16
    return 0
  lax.fori_loop(0, B, _z2, 0)
  def _z1(i, _):
    nbuf[pl.ds(i * L, L)] = zero16
    return 0
  lax.fori_loop(0, ROWS_PT // L, _z1, 0)
  pltpu.sync_copy(nbuf, deg_sh.at[pl.ds(s * ROWS_PT, ROWS_PT)])
  for b in range(ROWS_PT // B):
    pltpu.sync_copy(rows_v, acc_sh.at[pl.ds(s * ROWS_PT + b * B, B), :])
  def _o1(i, _):
    ewb[pl.ds(i * L, L)] = one16
    return 0
  lax.fori_loop(0, B // L, _o1, 0)
  for q in range(TAIL // L):
    ewb_t[pl.ds(q * L, L)] = one16
  plsc.subcore_barrier()

  e_base = s * EP

  def _p1(k, _):
    e0 = e_base + k * B
    pltpu.sync_copy(col_h.at[pl.ds(e0, B)], cidx)

    @pl.when(c == 0)
    def _():
      pltpu.sync_copy(ew_h.at[pl.ds(e0, B)], ewb)
    pltpu.sync_copy(ewb, deg_sh.at[cidx], add=True)
    return 0
  lax.fori_loop(0, NB, _p1, 0)
  e0t = e_base + NB * B
  pltpu.sync_copy(col_h.at[pl.ds(e0t, TAIL)], cidx_t)

  @pl.when(c == 0)
  def _():
    pltpu.sync_copy(ew_h.at[pl.ds(e0t, TAIL)], ewb_t)
  pltpu.sync_copy(ewb_t, deg_sh.at[cidx_t], add=True)
  plsc.subcore_barrier()

  n0 = s * ROWS_PT
  nrows = jnp.minimum(jnp.int32(ROWS_PT), jnp.int32(N) - n0)

  @pl.when(c == 1)
  def _():
    pltpu.sync_copy(deg_sh.at[pl.ds(n0, ROWS_PT)], nbuf)
    @pl.when(s < NS - 1)
    def _():
      pltpu.sync_copy(nbuf, cnt_h.at[pl.ds(n0, ROWS_PT)])
    @pl.when(s == NS - 1)
    def _():
      pltpu.sync_copy(nbuf.at[pl.ds(0, N - (NS - 1) * ROWS_PT)],
                      cnt_h.at[pl.ds(n0, N - (NS - 1) * ROWS_PT)])

  @pl.when(c == 0)
  def _():
    pltpu.sync_copy(deg_sh.at[pl.ds(n0, ROWS_PT)], nbuf)
    def _dv(i, _):
      d = nbuf[pl.ds(i * L, L)]
      nbuf[pl.ds(i * L, L)] = _rsqrt16(d)
      return 0
    lax.fori_loop(0, ROWS_PT // L, _dv, 0)
    pltpu.sync_copy(nbuf, dinv_sh.at[pl.ds(n0, ROWS_PT)])
    @pl.when(s < NS - 1)
    def _():
      pltpu.sync_copy(nbuf, dinv_h.at[pl.ds(n0, ROWS_PT)])
    @pl.when(s == NS - 1)
    def _():
      pltpu.sync_copy(nbuf.at[pl.ds(0, N - (NS - 1) * ROWS_PT)],
                      dinv_h.at[pl.ds(n0, N - (NS - 1) * ROWS_PT)])
    plsc.subcore_barrier()
    pltpu.sync_copy(dinv_sh, dloc)

  def _block(e0, idxb, cidxb, ewbuf, sclbuf, rbuf, nb):
    pltpu.sync_copy(row_h.at[pl.ds(e0, nb)], idxb)
    pltpu.sync_copy(col_h.at[pl.ds(e0, nb)], cidxb)
    pltpu.sync_copy(ew_h.at[pl.ds(e0, nb)], ewbuf)

    @pl.when(c == 0)
    def _():
      pltpu.async_copy(t_h.at[idxb], rbuf, sem).wait()
      for q in range(nb // L):
        r16 = idxb[pl.ds(q * L, L)]
        dv = plsc.load_gather(dloc, [r16])
        sclbuf[pl.ds(q * L, L)] = dv * ewbuf[pl.ds(q * L, L)]

    @pl.when(c == 1)
    def _():
      pltpu.async_copy(h_h.at[idxb], rbuf, sem).wait()
      for q in range(nb // L):
        sclbuf[pl.ds(q * L, L)] = ewbuf[pl.ds(q * L, L)]

    def _scale(i, _):
      sv = plsc.load_gather(sclbuf, [jnp.full((L,), i, jnp.int32)])
      for j in range(8):
        rbuf[i, pl.ds(j * L, L)] = rbuf[i, pl.ds(j * L, L)] * sv
      return 0
    lax.fori_loop(0, nb, _scale, 0)
    pltpu.sync_copy(rbuf, acc_sh.at[cidxb], add=True)

  def _p3(k, _):
    _block(e_base + k * B, ridx, cidx, ewb, scl, rows_v, B)
    return 0
  lax.fori_loop(0, NB, _p3, 0)
  _block(e_base + NB * B, ridx_t, cidx_t, ewb_t, scl_t, rows_t, TAIL)
  plsc.subcore_barrier()

  def _writeout(dst):
    nfull = ROWS_PT // B
    def _chunk(r0, nr):
      pltpu.sync_copy(acc_sh.at[pl.ds(r0, nr), :], rows_v.at[pl.ds(0, nr), :])
      pltpu.sync_copy(rows_v.at[pl.ds(0, nr), :], dst.at[pl.ds(r0, nr), :])
    @pl.when(s < NS - 1)
    def _():
      for b in range(nfull):
        _chunk(n0 + b * B, B)
    @pl.when(s == NS - 1)
    def _():
      nlast = N - (NS - 1) * ROWS_PT
      for b in range(nlast // B):
        _chunk(n0 + b * B, B)
      _chunk(n0 + (nlast // B) * B, nlast % B)

  @pl.when(c == 0)
  def _():
    _writeout(aggp_h)

  @pl.when(c == 1)
  def _():
    _writeout(aggs_h)


_sc_call = pl.kernel(
    _sc_body,
    out_type=(
        jax.ShapeDtypeStruct((N, OUT), jnp.float32),
        jax.ShapeDtypeStruct((N, HID), jnp.float32),
        jax.ShapeDtypeStruct((N,), jnp.float32),
        jax.ShapeDtypeStruct((N,), jnp.float32),
    ),
    mesh=plsc.VectorSubcoreMesh(core_axis_name="c", subcore_axis_name="s",
                                num_cores=NC, num_subcores=NS),
    compiler_params=pltpu.CompilerParams(needs_layout_passes=False),
    scratch_types=[
        pltpu.VMEM((B, 128), jnp.float32),
        pltpu.VMEM((TAIL, 128), jnp.float32),
        pltpu.VMEM((B,), jnp.int32),
        pltpu.VMEM((B,), jnp.int32),
        pltpu.VMEM((B,), jnp.float32),
        pltpu.VMEM((B,), jnp.float32),
        pltpu.VMEM((TAIL,), jnp.int32),
        pltpu.VMEM((TAIL,), jnp.int32),
        pltpu.VMEM((TAIL,), jnp.float32),
        pltpu.VMEM((TAIL,), jnp.float32),
        pltpu.VMEM((NPAD,), jnp.float32),
        pltpu.VMEM((ROWS_PT,), jnp.float32),
        pltpu.VMEM_SHARED((NPAD, 128), jnp.float32),
        pltpu.VMEM_SHARED((NPAD,), jnp.float32),
        pltpu.VMEM_SHARED((NPAD,), jnp.float32),
        pltpu.SemaphoreType.DMA,
    ],
)


_BN = 2000


def _pre_body(x_ref, wpt_ref, bp_ref, wi_ref, h_ref, t_ref):
  h = jnp.dot(x_ref[:], wpt_ref[:], preferred_element_type=jnp.float32)
  h = h + bp_ref[:]
  h_ref[:] = h
  t_ref[:] = jnp.dot(h, wi_ref[:], preferred_element_type=jnp.float32)


@functools.partial(jax.jit)
def _pre_call(x, wpt, bp, wi):
  return pl.pallas_call(
      _pre_body,
      grid=(N // _BN,),
      in_specs=[
          pl.BlockSpec((_BN, CUR), lambda i: (i, 0)),
          pl.BlockSpec((CUR, HID), lambda i: (0, 0)),
          pl.BlockSpec((1, HID), lambda i: (0, 0)),
          pl.BlockSpec((HID, OUT), lambda i: (0, 0)),
      ],
      out_specs=[
          pl.BlockSpec((_BN, HID), lambda i: (i, 0)),
          pl.BlockSpec((_BN, OUT), lambda i: (i, 0)),
      ],
      out_shape=[
          jax.ShapeDtypeStruct((N, HID), jnp.float32),
          jax.ShapeDtypeStruct((N, OUT), jnp.float32),
      ],
  )(x, wpt, bp, wi)


def _post_body(h_ref, aggp_ref, aggs_ref, cnt_ref, dinv_ref, wroot_ref,
               barma_ref, wlt_ref, bl_ref, wrt_ref, out_ref):
  h = h_ref[:]
  arma = dinv_ref[:] * aggp_ref[:] + jnp.dot(
      h, wroot_ref[:], preferred_element_type=jnp.float32) + barma_ref[:]
  arma = jnp.maximum(arma, 0.0)
  mean = aggs_ref[:] * (1.0 / jnp.maximum(cnt_ref[:], 1.0))
  sage = (jnp.dot(mean, wlt_ref[:], preferred_element_type=jnp.float32)
          + jnp.dot(h, wrt_ref[:], preferred_element_type=jnp.float32)
          + bl_ref[:])
  h2 = jnp.where(sage > 0.0, sage, jnp.exp(0.01 * sage) - 1.0)
  out_ref[:] = jnp.concatenate([arma, h2], axis=1)


@functools.partial(jax.jit)
def _post_call(h, aggp, aggs, cnt, dinv, wroot, barma, wlt, bl, wrt):
  return pl.pallas_call(
      _post_body,
      grid=(N // _BN,),
      in_specs=[
          pl.BlockSpec((_BN, HID), lambda i: (i, 0)),
          pl.BlockSpec((_BN, OUT), lambda i: (i, 0)),
          pl.BlockSpec((_BN, HID), lambda i: (i, 0)),
          pl.BlockSpec((_BN, 1), lambda i: (i, 0)),
          pl.BlockSpec((_BN, 1), lambda i: (i, 0)),
          pl.BlockSpec((HID, OUT), lambda i: (0, 0)),
          pl.BlockSpec((1, OUT), lambda i: (0, 0)),
          pl.BlockSpec((HID, OUT), lambda i: (0, 0)),
          pl.BlockSpec((1, OUT), lambda i: (0, 0)),
          pl.BlockSpec((HID, OUT), lambda i: (0, 0)),
      ],
      out_specs=pl.BlockSpec((_BN, 2 * OUT), lambda i: (i, 0)),
      out_shape=jax.ShapeDtypeStruct((N, 2 * OUT), jnp.float32),
  )(h, aggp, aggs, cnt, dinv, wroot, barma, wlt, bl, wrt)


def kernel(x, edge_index, edge_weight, Wp, bp, W_init, W_root, b_arma,
           W_l, b_l, W_r):
  row = edge_index[0]
  col = edge_index[1]
  h, t = _pre_call(x, Wp.T, bp[None, :], W_init)
  aggp, aggs, cnt, dinv = _sc_call(row, col, edge_weight, t, h)
  return _post_call(h, aggp, aggs, cnt[:, None], dinv[:, None],
                    W_root, b_arma[None, :], W_l.T, b_l[None, :], W_r.T)


# speedup vs baseline: 22.2894x; 2.8012x over previous
import functools

import jax
import jax.numpy as jnp
from jax import lax
from jax.experimental import pallas as pl
from jax.experimental.pallas import tpu as pltpu
from jax.experimental.pallas import tpu_sc as plsc

N, E, CUR, HID, OUT = 10000, 320000, 128, 128, 128
NC, NS, L = 2, 16, 16
NPAD = 10240
ROWS_PT = NPAD // NS
B = 128
NBT = E // B
NB = NBT // NS
XTRA = NBT - NB * NS
UNR = 12
NOUT = NB // UNR

_MESH = plsc.VectorSubcoreMesh(core_axis_name="c", subcore_axis_name="s",
                               num_cores=NC, num_subcores=NS)
_SC_PARAMS = pltpu.CompilerParams(needs_layout_passes=False)


def _edge_split(s):
  e_base = (s * NB + jnp.minimum(s, XTRA)) * B
  return e_base, s < XTRA, e_base + NB * B


def _sc1_body(col_h, ew_h, deg_h, cnt_h,
              cib, ewb, ones_v, nbuf, deg_sh, si0, si1, si2, si3, sp0, sp1):
  c = lax.axis_index("c")
  s = lax.axis_index("s")
  zero16 = jnp.zeros((L,), jnp.float32)
  one16 = jnp.ones((L,), jnp.float32)
  semI = (si0, si1, si2, si3)
  semP = (sp0, sp1)
  e_base, has_x, e_x = _edge_split(s)
  n0 = s * ROWS_PT
  nlast = N - (NS - 1) * ROWS_PT

  def _d_col(e0, slot):
    return pltpu.make_async_copy(col_h.at[pl.ds(e0, B)], cib.at[slot],
                                 semI[slot])

  def _d_ew(e0, slot):
    return pltpu.make_async_copy(ew_h.at[pl.ds(e0, B)],
                                 ewb.at[pl.ds(slot * B, B)], semI[slot])

  def _src(slot, with_ew):
    return ewb.at[pl.ds(slot * B, B)] if with_ew else ones_v

  def _d_deg(slot2, slot4, with_ew):
    return pltpu.make_async_copy(_src(slot4, with_ew),
                                 deg_sh.at[cib.at[slot4]], semP[slot2])

  def _z1(i, _):
    nbuf[pl.ds(i * L, L)] = zero16
    return 0
  lax.fori_loop(0, ROWS_PT // L, _z1, 0)
  pltpu.sync_copy(nbuf, deg_sh.at[pl.ds(n0, ROWS_PT)])
  for q in range(B // L):
    ones_v[pl.ds(q * L, L)] = one16
  plsc.subcore_barrier()

  def _sweep(with_ew):
    for slot, e0 in ((0, e_base), (1, e_base + B)):
      _d_col(e0, slot).start()
      if with_ew:
        _d_ew(e0, slot).start()

    def _p1(o, _):
      for j in range(4):
        k = o * 4 + j
        e0 = e_base + k * B
        @pl.when(k < NB - 2)
        def _():
          _d_col(e0 + 2 * B, (j + 2) % 4).start()
          if with_ew:
            _d_ew(e0 + 2 * B, (j + 2) % 4).start()
        _d_col(e0, j).wait()
        if with_ew:
          _d_ew(e0, j).wait()
        @pl.when(k >= 1)
        def _():
          _d_deg((j + 1) % 2, (j + 3) % 4, with_ew).wait()
        pltpu.async_copy(_src(j, with_ew), deg_sh.at[cib.at[j]],
                         semP[j % 2], add=True)
      return 0
    lax.fori_loop(0, NB // 4, _p1, 0)
    _d_deg(1, 3, with_ew).wait()
    @pl.when(has_x)
    def _():
      pltpu.sync_copy(col_h.at[pl.ds(e_x, B)], cib.at[0])
      if with_ew:
        pltpu.sync_copy(ew_h.at[pl.ds(e_x, B)], ewb.at[pl.ds(0, B)])
      pltpu.sync_copy(_src(0, with_ew), deg_sh.at[cib.at[0]], add=True)

  @pl.when(c == 0)
  def _():
    _sweep(True)

  @pl.when(c == 1)
  def _():
    _sweep(False)
  plsc.subcore_barrier()

  def _wout(dst):
    pltpu.sync_copy(deg_sh.at[pl.ds(n0, ROWS_PT)], nbuf)
    @pl.when(s < NS - 1)
    def _():
      pltpu.sync_copy(nbuf, dst.at[pl.ds(n0, ROWS_PT)])
    @pl.when(s == NS - 1)
    def _():
      pltpu.sync_copy(nbuf.at[pl.ds(0, nlast)], dst.at[pl.ds(n0, nlast)])

  @pl.when(c == 0)
  def _():
    _wout(deg_h)

  @pl.when(c == 1)
  def _():
    _wout(cnt_h)


_sc1_call = pl.kernel(
    _sc1_body,
    out_type=(
        jax.ShapeDtypeStruct((N,), jnp.float32),
        jax.ShapeDtypeStruct((N,), jnp.float32),
    ),
    mesh=_MESH,
    compiler_params=_SC_PARAMS,
    scratch_types=[
        pltpu.VMEM((4, B), jnp.int32),
        pltpu.VMEM((4 * B,), jnp.float32),
        pltpu.VMEM((B,), jnp.float32),
        pltpu.VMEM((ROWS_PT,), jnp.float32),
        pltpu.VMEM_SHARED((NPAD,), jnp.float32),
    ] + [pltpu.SemaphoreType.DMA] * 6,
)


def _sc2_body(row_h, col_h, ew_h, t_h, h_h, aggp_h, aggs_h,
              rib, cib, ewb, rows, acc_sh,
              si0, si1, si2, si3, sg0, sg1, sg2, ss0, ss1, ss2):
  c = lax.axis_index("c")
  s = lax.axis_index("s")
  zero16 = jnp.zeros((L,), jnp.float32)
  semI = (si0, si1, si2, si3)
  semG = (sg0, sg1, sg2)
  semS = (ss0, ss1, ss2)
  e_base, has_x, e_x = _edge_split(s)
  n0 = s * ROWS_PT
  nlast = N - (NS - 1) * ROWS_PT

  def _d_row(e0, slot4, slot3):
    return pltpu.make_async_copy(row_h.at[pl.ds(e0, B)],
                                 rib.at[pl.ds(slot3 * B, B)], semI[slot4])

  def _d_col(e0, slot4):
    return pltpu.make_async_copy(col_h.at[pl.ds(e0, B)], cib.at[slot4],
                                 semI[slot4])

  def _d_ew(e0, slot4, slot3):
    return pltpu.make_async_copy(ew_h.at[pl.ds(e0, B)],
                                 ewb.at[pl.ds(slot3 * B, B)], semI[slot4])

  def _d_gath(tab, slot3):
    return pltpu.make_async_copy(tab.at[rib.at[pl.ds(slot3 * B, B)]],
                                 rows.at[pl.ds(slot3 * B, B), :], semG[slot3])

  def _d_scat(slot3, slot4):
    return pltpu.make_async_copy(rows.at[pl.ds(slot3 * B, B), :],
                                 acc_sh.at[cib.at[slot4]], semS[slot3])

  def _z2(i, _):
    for j in range(8):
      rows[i, pl.ds(j * L, L)] = zero16
    return 0
  lax.fori_loop(0, B, _z2, 0)
  zrows = rows.at[pl.ds(0, B), :]
  @pl.when(s < NS - 1)
  def _():
    for b in range(ROWS_PT // B):
      pltpu.sync_copy(zrows, acc_sh.at[pl.ds(n0 + b * B, B), :])
  @pl.when(s == NS - 1)
  def _():
    for b in range(nlast // B):
      pltpu.sync_copy(zrows, acc_sh.at[pl.ds(n0 + b * B, B), :])
    pltpu.sync_copy(rows.at[pl.ds(0, nlast % B), :],
                    acc_sh.at[pl.ds(n0 + (nlast // B) * B, nlast % B), :])
  plsc.subcore_barrier()

  def _mul_block(slot3):
    def _mul2(ii, _):
      for u in range(2):
        i = 2 * ii + u
        sv = plsc.load_gather(ewb, [jnp.full((L,), slot3 * B + i, jnp.int32)])
        for jf in range(8):
          rows[slot3 * B + i, pl.ds(jf * L, L)] = (
              rows[slot3 * B + i, pl.ds(jf * L, L)] * sv)
      return 0
    lax.fori_loop(0, B // 2, _mul2, 0)

  def _main(tab):
    for bk, e0 in ((0, e_base), (1, e_base + B)):
      _d_row(e0, bk, bk).start()
      _d_col(e0, bk).start()
      _d_ew(e0, bk, bk).start()
    _d_row(e_base, 0, 0).wait()
    _d_gath(tab, 0).start()

    def _mn(o, _):
      for j in range(UNR):
        k = o * UNR + j
        e0 = e_base + k * B
        s3, s4 = j % 3, j % 4
        @pl.when(k >= 2)
        def _():
          _d_scat((j + 1) % 3, (j + 2) % 4).wait()
        @pl.when(k < NB - 2)
        def _():
          _d_row(e0 + 2 * B, (j + 2) % 4, (j + 2) % 3).start()
          _d_col(e0 + 2 * B, (j + 2) % 4).start()
          _d_ew(e0 + 2 * B, (j + 2) % 4, (j + 2) % 3).start()
        @pl.when(k < NB - 1)
        def _():
          _d_row(e0 + B, (j + 1) % 4, (j + 1) % 3).wait()
          _d_gath(tab, (j + 1) % 3).start()
        _d_gath(tab, s3).wait()
        _d_col(e0, s4).wait()
        _d_ew(e0, s4, s3).wait()
        _mul_block(s3)
        pltpu.async_copy(rows.at[pl.ds(s3 * B, B), :],
                         acc_sh.at[cib.at[s4]], semS[s3], add=True)
      return 0
    lax.fori_loop(0, NOUT, _mn, 0)
    for (s3, s4) in (((UNR - 2) % 3, (UNR - 2) % 4),
                     ((UNR - 1) % 3, (UNR - 1) % 4)):
      _d_scat(s3, s4).wait()
    @pl.when(has_x)
    def _():
      pltpu.sync_copy(row_h.at[pl.ds(e_x, B)], rib.at[pl.ds(0, B)])
      pltpu.sync_copy(col_h.at[pl.ds(e_x, B)], cib.at[0])
      pltpu.sync_copy(ew_h.at[pl.ds(e_x, B)], ewb.at[pl.ds(0, B)])
      pltpu.async_copy(tab.at[rib.at[pl.ds(0, B)]],
                       rows.at[pl.ds(0, B), :], sg0).wait()
      _mul_block(0)
      pltpu.sync_copy(rows.at[pl.ds(0, B), :], acc_sh.at[cib.at[0]], add=True)

  @pl.when(c == 0)
  def _():
    _main(t_h)

  @pl.when(c == 1)
  def _():
    _main(h_h)
  plsc.subcore_barrier()

  def _writeout(dst):
    def _chunk(r0, nr, b3):
      pltpu.sync_copy(acc_sh.at[pl.ds(r0, nr), :],
                      rows.at[pl.ds(b3 * B, nr), :])
      pltpu.sync_copy(rows.at[pl.ds(b3 * B, nr), :], dst.at[pl.ds(r0, nr), :])
    @pl.when(s < NS - 1)
    def _():
      for b in range(ROWS_PT // B):
        _chunk(n0 + b * B, B, b % 3)
    @pl.when(s == NS - 1)
    def _():
      for b in range(nlast // B):
        _chunk(n0 + b * B, B, b % 3)
      _chunk(n0 + (nlast // B) * B, nlast % B, (nlast // B) % 3)

  @pl.when(c == 0)
  def _():
    _writeout(aggp_h)

  @pl.when(c == 1)
  def _():
    _writeout(aggs_h)


_sc2_call = pl.kernel(
    _sc2_body,
    out_type=(
        jax.ShapeDtypeStruct((N, OUT), jnp.float32),
        jax.ShapeDtypeStruct((N, HID), jnp.float32),
    ),
    mesh=_MESH,
    compiler_params=_SC_PARAMS,
    scratch_types=[
        pltpu.VMEM((3 * B,), jnp.int32),
        pltpu.VMEM((4, B), jnp.int32),
        pltpu.VMEM((3 * B,), jnp.float32),
        pltpu.VMEM((3 * B, 128), jnp.float32),
        pltpu.VMEM_SHARED((N, 128), jnp.float32),
    ] + [pltpu.SemaphoreType.DMA] * 10,
)


_BN = 2000


def _pre_body(x_ref, wpt_ref, bp_ref, wi_ref, deg_ref, cnt_ref,
              h_ref, t_ref, dinv_ref, icnt_ref):
  h = jnp.dot(x_ref[:], wpt_ref[:], preferred_element_type=jnp.float32)
  h = h + bp_ref[:]
  h_ref[:] = h
  deg = deg_ref[:]
  dinv = jnp.where(deg > 0.0, lax.rsqrt(jnp.maximum(deg, 1e-30)), 0.0)
  dinv_ref[:] = dinv
  icnt_ref[:] = 1.0 / jnp.maximum(cnt_ref[:], 1.0)
  t_ref[:] = dinv * jnp.dot(h, wi_ref[:], preferred_element_type=jnp.float32)


@functools.partial(jax.jit)
def _pre_call(x, wpt, bp, wi, deg, cnt):
  return pl.pallas_call(
      _pre_body,
      grid=(N // _BN,),
      in_specs=[
          pl.BlockSpec((_BN, CUR), lambda i: (i, 0)),
          pl.BlockSpec((CUR, HID), lambda i: (0, 0)),
          pl.BlockSpec((1, HID), lambda i: (0, 0)),
          pl.BlockSpec((HID, OUT), lambda i: (0, 0)),
          pl.BlockSpec((_BN, 1), lambda i: (i, 0)),
          pl.BlockSpec((_BN, 1), lambda i: (i, 0)),
      ],
      out_specs=[
          pl.BlockSpec((_BN, HID), lambda i: (i, 0)),
          pl.BlockSpec((_BN, OUT), lambda i: (i, 0)),
          pl.BlockSpec((_BN, 1), lambda i: (i, 0)),
          pl.BlockSpec((_BN, 1), lambda i: (i, 0)),
      ],
      out_shape=[
          jax.ShapeDtypeStruct((N, HID), jnp.float32),
          jax.ShapeDtypeStruct((N, OUT), jnp.float32),
          jax.ShapeDtypeStruct((N, 1), jnp.float32),
          jax.ShapeDtypeStruct((N, 1), jnp.float32),
      ],
  )(x, wpt, bp, wi, deg, cnt)


def _post_body(h_ref, aggp_ref, aggs_ref, icnt_ref, dinv_ref, wroot_ref,
               barma_ref, wlt_ref, bl_ref, wrt_ref, out_ref):
  h = h_ref[:]
  arma = dinv_ref[:] * aggp_ref[:] + jnp.dot(
      h, wroot_ref[:], preferred_element_type=jnp.float32) + barma_ref[:]
  arma = jnp.maximum(arma, 0.0)
  mean = aggs_ref[:] * icnt_ref[:]
  sage = (jnp.dot(mean, wlt_ref[:], preferred_element_type=jnp.float32)
          + jnp.dot(h, wrt_ref[:], preferred_element_type=jnp.float32)
          + bl_ref[:])
  h2 = jnp.where(sage > 0.0, sage, jnp.exp(0.01 * sage) - 1.0)
  out_ref[:] = jnp.concatenate([arma, h2], axis=1)


@functools.partial(jax.jit)
def _post_call(h, aggp, aggs, icnt, dinv, wroot, barma, wlt, bl, wrt):
  return pl.pallas_call(
      _post_body,
      grid=(N // _BN,),
      in_specs=[
          pl.BlockSpec((_BN, HID), lambda i: (i, 0)),
          pl.BlockSpec((_BN, OUT), lambda i: (i, 0)),
          pl.BlockSpec((_BN, HID), lambda i: (i, 0)),
          pl.BlockSpec((_BN, 1), lambda i: (i, 0)),
          pl.BlockSpec((_BN, 1), lambda i: (i, 0)),
          pl.BlockSpec((HID, OUT), lambda i: (0, 0)),
          pl.BlockSpec((1, OUT), lambda i: (0, 0)),
          pl.BlockSpec((HID, OUT), lambda i: (0, 0)),
          pl.BlockSpec((1, OUT), lambda i: (0, 0)),
          pl.BlockSpec((HID, OUT), lambda i: (0, 0)),
      ],
      out_specs=pl.BlockSpec((_BN, 2 * OUT), lambda i: (i, 0)),
      out_shape=jax.ShapeDtypeStruct((N, 2 * OUT), jnp.float32),
  )(h, aggp, aggs, icnt, dinv, wroot, barma, wlt, bl, wrt)


def kernel(x, edge_index, edge_weight, Wp, bp, W_init, W_root, b_arma,
           W_l, b_l, W_r):
  row = edge_index[0]
  col = edge_index[1]
  deg, cnt = _sc1_call(col, edge_weight)
  h, t, dinv, icnt = _pre_call(x, Wp.T, bp[None, :], W_init,
                               deg[:, None], cnt[:, None])
  aggp, aggs = _sc2_call(row, col, edge_weight, t, h)
  return _post_call(h, aggp, aggs, icnt, dinv,
                    W_root, b_arma[None, :], W_l.T, b_l[None, :], W_r.T)
